# baseline (speedup 1.0000x reference)
import math

import jax
import jax.numpy as jnp
from jax.experimental import pallas as pl
from jax.experimental.pallas import tpu as pltpu

N_Q = 8
BINS = 1024
DIM = 128
HOP = 320
COMMIT_W = 0.1

TT = 512


def _rvq_body(x_ref, cb_ref, cbt_ref,
              quant_ref, codes_ref, subq_ref, dist_ref, closs_ref):
    r = x_ref[...]
    qacc = jnp.zeros_like(r)
    cacc = jnp.zeros((1, N_Q), jnp.float32)
    qpos = jax.lax.broadcasted_iota(jnp.int32, (1, N_Q), 1)

    @pl.when(pl.program_id(0) == 0)
    def _():
        closs_ref[...] = jnp.zeros_like(closs_ref)

    for q in range(N_Q):
        emb = cb_ref[q]
        embt = cbt_ref[q]
        e2 = jnp.sum(emb * emb, axis=1)[None, :]
        r2 = jnp.sum(r * r, axis=1, keepdims=True)
        s = jnp.dot(r.astype(jnp.bfloat16), embt.astype(jnp.bfloat16),
                    preferred_element_type=jnp.float32)
        dist = -(r2 - 2.0 * s + e2)
        idx = jnp.argmax(dist, axis=1)
        oh = (jax.lax.broadcasted_iota(jnp.int32, (TT, BINS), 1)
              == idx[:, None]).astype(jnp.float32)
        quant = jnp.dot(oh, emb, precision=jax.lax.Precision.HIGHEST,
                        preferred_element_type=jnp.float32)
        diff = quant - r
        cacc = cacc + jnp.sum(diff * diff) * (qpos == q).astype(jnp.float32)
        qacc = qacc + quant
        r = r - quant
        codes_ref[q, 0, :, :] = idx[None, :]
        subq_ref[q, 0] = quant
        if q == N_Q - 1:
            dist_ref[...] = dist
    closs_ref[...] += cacc
    quant_ref[...] = qacc


def kernel(x, sample_rate, codebooks):
    B, C, T = x.shape
    NTOK = B * T
    NT = NTOK // TT

    flat = jnp.transpose(x, (0, 2, 1)).reshape(NTOK, C)
    cbt = jnp.transpose(codebooks, (0, 2, 1))

    grid = (NT,)
    out_shapes = (
        jax.ShapeDtypeStruct((NTOK, DIM), jnp.float32),
        jax.ShapeDtypeStruct((N_Q, NT, 1, TT), jnp.int32),
        jax.ShapeDtypeStruct((N_Q, NT, TT, DIM), jnp.float32),
        jax.ShapeDtypeStruct((NTOK, BINS), jnp.float32),
        jax.ShapeDtypeStruct((1, N_Q), jnp.float32),
    )
    in_specs = [
        pl.BlockSpec((TT, DIM), lambda i: (i, 0)),
        pl.BlockSpec((N_Q, BINS, DIM), lambda i: (0, 0, 0)),
        pl.BlockSpec((N_Q, DIM, BINS), lambda i: (0, 0, 0)),
    ]
    out_specs = (
        pl.BlockSpec((TT, DIM), lambda i: (i, 0)),
        pl.BlockSpec((N_Q, 1, 1, TT), lambda i: (0, i, 0, 0)),
        pl.BlockSpec((N_Q, 1, TT, DIM), lambda i: (0, i, 0, 0)),
        pl.BlockSpec((TT, BINS), lambda i: (i, 0)),
        pl.BlockSpec((1, N_Q), lambda i: (0, 0)),
    )
    quant_flat, codes_raw, subq_raw, last_dist, closs_raw = pl.pallas_call(
        _rvq_body,
        grid=grid,
        in_specs=in_specs,
        out_specs=out_specs,
        out_shape=out_shapes,
        compiler_params=pltpu.CompilerParams(
            dimension_semantics=("arbitrary",),
        ),
    )(flat, codebooks, cbt)

    quantized = jnp.transpose(quant_flat.reshape(B, T, C), (0, 2, 1))
    codes = codes_raw.reshape(N_Q, B, T)
    sub_quants = jnp.transpose(subq_raw.reshape(N_Q, B, T, C), (0, 1, 3, 2))
    commit_loss = (COMMIT_W / (NTOK * C)) * closs_raw[0]
    penalty = jnp.sum(commit_loss) / jnp.maximum(jnp.sum(commit_loss != 0), 1)
    bw_per_q = math.log2(BINS) * jnp.asarray(sample_rate, dtype=jnp.float32) / HOP
    bw = (N_Q * bw_per_q).astype(x.dtype)
    return (quantized, codes, bw, penalty, sub_quants, last_dist)


# speedup vs baseline: 1.5642x; 1.5642x over previous
import math

import jax
import jax.numpy as jnp
from jax.experimental import pallas as pl
from jax.experimental.pallas import tpu as pltpu

N_Q = 8
BINS = 1024
DIM = 128
HOP = 320
COMMIT_W = 0.1

TT = 512


def _rvq_body(x_ref, cb_ref, cbt_ref, ehi_ref, emid_ref, elo_ref,
              quant_ref, codes_ref, subq_ref, dist_ref, closs_ref):
    r = x_ref[...]
    qacc = jnp.zeros_like(r)
    cacc = jnp.zeros((1, N_Q), jnp.float32)
    qpos = jax.lax.broadcasted_iota(jnp.int32, (1, N_Q), 1)
    iota_b = jax.lax.broadcasted_iota(jnp.int32, (TT, BINS), 1)

    @pl.when(pl.program_id(0) == 0)
    def _():
        closs_ref[...] = jnp.zeros_like(closs_ref)

    for q in range(N_Q):
        emb = cb_ref[q]
        e2 = jnp.sum(emb * emb, axis=1)[None, :]
        r2 = jnp.sum(r * r, axis=1, keepdims=True)
        rb = r.astype(jnp.bfloat16)
        s = jnp.dot(rb, cbt_ref[q], preferred_element_type=jnp.float32)
        t = r2 - 2.0 * s + e2
        idx = jnp.argmax(-t, axis=1)
        oh = (iota_b == idx[:, None]).astype(jnp.float32)
        quant = (jnp.dot(oh, ehi_ref[q], preferred_element_type=jnp.float32)
                 + jnp.dot(oh, emid_ref[q], preferred_element_type=jnp.float32)
                 + jnp.dot(oh, elo_ref[q], preferred_element_type=jnp.float32))
        diff = quant - r
        cacc = cacc + jnp.sum(diff * diff) * (qpos == q).astype(jnp.float32)
        qacc = qacc + quant
        r = r - quant
        codes_ref[q, 0, :, :] = idx[None, :]
        subq_ref[q, 0] = quant
        if q == N_Q - 1:
            dist_ref[...] = -t
    closs_ref[...] += cacc
    quant_ref[...] = qacc


def _trunc_bf16(a):
    return jax.lax.bitcast_convert_type(
        jax.lax.bitcast_convert_type(a, jnp.uint32) & jnp.uint32(0xFFFF0000),
        jnp.float32)


def _split3(a):
    hi = _trunc_bf16(a)
    rem = a - hi
    mid = _trunc_bf16(rem)
    lo = rem - mid
    return hi, mid, lo


def kernel(x, sample_rate, codebooks):
    B, C, T = x.shape
    NTOK = B * T
    NT = NTOK // TT

    flat = jnp.transpose(x, (0, 2, 1)).reshape(NTOK, C)
    cbt = jnp.transpose(codebooks, (0, 2, 1)).astype(jnp.bfloat16)
    ehi, emid, elo = _split3(codebooks)

    grid = (NT,)
    out_shapes = (
        jax.ShapeDtypeStruct((NTOK, DIM), jnp.float32),
        jax.ShapeDtypeStruct((N_Q, NT, 1, TT), jnp.int32),
        jax.ShapeDtypeStruct((N_Q, NT, TT, DIM), jnp.float32),
        jax.ShapeDtypeStruct((NTOK, BINS), jnp.float32),
        jax.ShapeDtypeStruct((1, N_Q), jnp.float32),
    )
    cb_spec = pl.BlockSpec((N_Q, BINS, DIM), lambda i: (0, 0, 0))
    in_specs = [
        pl.BlockSpec((TT, DIM), lambda i: (i, 0)),
        cb_spec,
        pl.BlockSpec((N_Q, DIM, BINS), lambda i: (0, 0, 0)),
        cb_spec, cb_spec, cb_spec,
    ]
    out_specs = (
        pl.BlockSpec((TT, DIM), lambda i: (i, 0)),
        pl.BlockSpec((N_Q, 1, 1, TT), lambda i: (0, i, 0, 0)),
        pl.BlockSpec((N_Q, 1, TT, DIM), lambda i: (0, i, 0, 0)),
        pl.BlockSpec((TT, BINS), lambda i: (i, 0)),
        pl.BlockSpec((1, N_Q), lambda i: (0, 0)),
    )
    quant_flat, codes_raw, subq_raw, last_dist, closs_raw = pl.pallas_call(
        _rvq_body,
        grid=grid,
        in_specs=in_specs,
        out_specs=out_specs,
        out_shape=out_shapes,
        compiler_params=pltpu.CompilerParams(
            dimension_semantics=("arbitrary",),
        ),
    )(flat, codebooks, cbt, ehi, emid, elo)

    quantized = jnp.transpose(quant_flat.reshape(B, T, C), (0, 2, 1))
    codes = codes_raw.reshape(N_Q, B, T)
    sub_quants = jnp.transpose(subq_raw.reshape(N_Q, B, T, C), (0, 1, 3, 2))
    commit_loss = (COMMIT_W / (NTOK * C)) * closs_raw[0]
    penalty = jnp.sum(commit_loss) / jnp.maximum(jnp.sum(commit_loss != 0), 1)
    bw_per_q = math.log2(BINS) * jnp.asarray(sample_rate, dtype=jnp.float32) / HOP
    bw = (N_Q * bw_per_q).astype(x.dtype)
    return (quantized, codes, bw, penalty, sub_quants, last_dist)


# speedup vs baseline: 2.5919x; 1.6570x over previous
import math

import jax
import jax.numpy as jnp
from jax.experimental import pallas as pl
from jax.experimental.pallas import tpu as pltpu

N_Q = 8
BINS = 1024
DIM = 128
HOP = 320
COMMIT_W = 0.1

TT = 512


def _rvq_body(x_ref, cb_ref, cbt_ref, e3_ref,
              quant_ref, codes_ref, subq_ref, dist_ref, closs_ref):
    r = x_ref[...]
    qacc = jnp.zeros_like(r)
    cacc = jnp.zeros((1, N_Q), jnp.float32)
    qpos = jax.lax.broadcasted_iota(jnp.int32, (1, N_Q), 1)
    iota_b = jax.lax.broadcasted_iota(jnp.int32, (TT, BINS), 1)

    @pl.when(pl.program_id(0) == 0)
    def _():
        closs_ref[...] = jnp.zeros_like(closs_ref)

    for q in range(N_Q):
        emb = cb_ref[q]
        e2 = jnp.sum(emb * emb, axis=1)[None, :]
        r2 = jnp.sum(r * r, axis=1, keepdims=True)
        rb2 = (r + r).astype(jnp.bfloat16)
        s2 = jnp.dot(rb2, cbt_ref[q], preferred_element_type=jnp.float32)
        t = r2 - s2 + e2
        m = jnp.min(t, axis=1, keepdims=True)
        idx = jnp.min(jnp.where(t == m, iota_b, BINS), axis=1)
        oh = (iota_b == idx[:, None]).astype(jnp.float32)
        q3 = jnp.dot(oh, e3_ref[q], preferred_element_type=jnp.float32)
        quant = (q3[:, :DIM] + q3[:, DIM:2 * DIM]) + q3[:, 2 * DIM:]
        diff = quant - r
        cacc = cacc + jnp.sum(diff * diff) * (qpos == q).astype(jnp.float32)
        qacc = qacc + quant
        r = r - quant
        codes_ref[q, 0, :, :] = idx[None, :]
        subq_ref[q, 0] = quant
        if q == N_Q - 1:
            dist_ref[...] = -t
    closs_ref[...] += cacc
    quant_ref[...] = qacc


def _trunc_bf16(a):
    return jax.lax.bitcast_convert_type(
        jax.lax.bitcast_convert_type(a, jnp.uint32) & jnp.uint32(0xFFFF0000),
        jnp.float32)


def _split3(a):
    hi = _trunc_bf16(a)
    rem = a - hi
    mid = _trunc_bf16(rem)
    lo = rem - mid
    return hi, mid, lo


def kernel(x, sample_rate, codebooks):
    B, C, T = x.shape
    NTOK = B * T
    NT = NTOK // TT

    flat = jnp.transpose(x, (0, 2, 1)).reshape(NTOK, C)
    cbt = jnp.transpose(codebooks, (0, 2, 1)).astype(jnp.bfloat16)
    e3 = jnp.concatenate(_split3(codebooks), axis=-1)

    grid = (NT,)
    out_shapes = (
        jax.ShapeDtypeStruct((NTOK, DIM), jnp.float32),
        jax.ShapeDtypeStruct((N_Q, NT, 1, TT), jnp.int32),
        jax.ShapeDtypeStruct((N_Q, NT, TT, DIM), jnp.float32),
        jax.ShapeDtypeStruct((NTOK, BINS), jnp.float32),
        jax.ShapeDtypeStruct((1, N_Q), jnp.float32),
    )
    in_specs = [
        pl.BlockSpec((TT, DIM), lambda i: (i, 0)),
        pl.BlockSpec((N_Q, BINS, DIM), lambda i: (0, 0, 0)),
        pl.BlockSpec((N_Q, DIM, BINS), lambda i: (0, 0, 0)),
        pl.BlockSpec((N_Q, BINS, 3 * DIM), lambda i: (0, 0, 0)),
    ]
    out_specs = (
        pl.BlockSpec((TT, DIM), lambda i: (i, 0)),
        pl.BlockSpec((N_Q, 1, 1, TT), lambda i: (0, i, 0, 0)),
        pl.BlockSpec((N_Q, 1, TT, DIM), lambda i: (0, i, 0, 0)),
        pl.BlockSpec((TT, BINS), lambda i: (i, 0)),
        pl.BlockSpec((1, N_Q), lambda i: (0, 0)),
    )
    quant_flat, codes_raw, subq_raw, last_dist, closs_raw = pl.pallas_call(
        _rvq_body,
        grid=grid,
        in_specs=in_specs,
        out_specs=out_specs,
        out_shape=out_shapes,
        compiler_params=pltpu.CompilerParams(
            dimension_semantics=("arbitrary",),
        ),
    )(flat, codebooks, cbt, e3)

    quantized = jnp.transpose(quant_flat.reshape(B, T, C), (0, 2, 1))
    codes = codes_raw.reshape(N_Q, B, T)
    sub_quants = jnp.transpose(subq_raw.reshape(N_Q, B, T, C), (0, 1, 3, 2))
    commit_loss = (COMMIT_W / (NTOK * C)) * closs_raw[0]
    penalty = jnp.sum(commit_loss) / jnp.maximum(jnp.sum(commit_loss != 0), 1)
    bw_per_q = math.log2(BINS) * jnp.asarray(sample_rate, dtype=jnp.float32) / HOP
    bw = (N_Q * bw_per_q).astype(x.dtype)
    return (quantized, codes, bw, penalty, sub_quants, last_dist)


# speedup vs baseline: 2.8293x; 1.0916x over previous
import math

import jax
import jax.numpy as jnp
from jax.experimental import pallas as pl
from jax.experimental.pallas import tpu as pltpu

N_Q = 8
BINS = 1024
DIM = 128
HOP = 320
COMMIT_W = 0.1

TT = 1024


def _rvq_body(x_ref, cb_ref, cbt_ref, e3_ref,
              quant_ref, codes_ref, subq_ref, dist_ref, closs_ref):
    r = x_ref[...]
    qacc = jnp.zeros_like(r)
    cacc = jnp.zeros((1, N_Q), jnp.float32)
    qpos = jax.lax.broadcasted_iota(jnp.int32, (1, N_Q), 1)
    iota_b = jax.lax.broadcasted_iota(jnp.int32, (TT, BINS), 1)

    @pl.when(pl.program_id(0) == 0)
    def _():
        closs_ref[...] = jnp.zeros_like(closs_ref)

    for q in range(N_Q):
        emb = cb_ref[q]
        e2 = jnp.sum(emb * emb, axis=1)[None, :]
        r2 = jnp.sum(r * r, axis=1, keepdims=True)
        rb2 = (r + r).astype(jnp.bfloat16)
        s2 = jnp.dot(rb2, cbt_ref[q], preferred_element_type=jnp.float32)
        t = r2 - s2 + e2
        m = jnp.min(t, axis=1, keepdims=True)
        idx = jnp.min(jnp.where(t == m, iota_b, BINS), axis=1)
        oh = (iota_b == idx[:, None]).astype(jnp.float32)
        q3 = jnp.dot(oh, e3_ref[q], preferred_element_type=jnp.float32)
        quant = (q3[:, :DIM] + q3[:, DIM:2 * DIM]) + q3[:, 2 * DIM:]
        diff = quant - r
        cacc = cacc + jnp.sum(diff * diff) * (qpos == q).astype(jnp.float32)
        qacc = qacc + quant
        r = r - quant
        codes_ref[q, 0, :, :] = idx[None, :]
        subq_ref[q, 0] = quant
        if q == N_Q - 1:
            dist_ref[...] = -t
    closs_ref[...] += cacc
    quant_ref[...] = qacc


def _trunc_bf16(a):
    return jax.lax.bitcast_convert_type(
        jax.lax.bitcast_convert_type(a, jnp.uint32) & jnp.uint32(0xFFFF0000),
        jnp.float32)


def _split3(a):
    hi = _trunc_bf16(a)
    rem = a - hi
    mid = _trunc_bf16(rem)
    lo = rem - mid
    return hi, mid, lo


def kernel(x, sample_rate, codebooks):
    B, C, T = x.shape
    NTOK = B * T
    NT = NTOK // TT

    flat = jnp.transpose(x, (0, 2, 1)).reshape(NTOK, C)
    cbt = jnp.transpose(codebooks, (0, 2, 1)).astype(jnp.bfloat16)
    e3 = jnp.concatenate(_split3(codebooks), axis=-1)

    grid = (NT,)
    out_shapes = (
        jax.ShapeDtypeStruct((NTOK, DIM), jnp.float32),
        jax.ShapeDtypeStruct((N_Q, NT, 1, TT), jnp.int32),
        jax.ShapeDtypeStruct((N_Q, NT, TT, DIM), jnp.float32),
        jax.ShapeDtypeStruct((NTOK, BINS), jnp.float32),
        jax.ShapeDtypeStruct((1, N_Q), jnp.float32),
    )
    in_specs = [
        pl.BlockSpec((TT, DIM), lambda i: (i, 0)),
        pl.BlockSpec((N_Q, BINS, DIM), lambda i: (0, 0, 0)),
        pl.BlockSpec((N_Q, DIM, BINS), lambda i: (0, 0, 0)),
        pl.BlockSpec((N_Q, BINS, 3 * DIM), lambda i: (0, 0, 0)),
    ]
    out_specs = (
        pl.BlockSpec((TT, DIM), lambda i: (i, 0)),
        pl.BlockSpec((N_Q, 1, 1, TT), lambda i: (0, i, 0, 0)),
        pl.BlockSpec((N_Q, 1, TT, DIM), lambda i: (0, i, 0, 0)),
        pl.BlockSpec((TT, BINS), lambda i: (i, 0)),
        pl.BlockSpec((1, N_Q), lambda i: (0, 0)),
    )
    quant_flat, codes_raw, subq_raw, last_dist, closs_raw = pl.pallas_call(
        _rvq_body,
        grid=grid,
        in_specs=in_specs,
        out_specs=out_specs,
        out_shape=out_shapes,
        compiler_params=pltpu.CompilerParams(
            dimension_semantics=("arbitrary",),
        ),
    )(flat, codebooks, cbt, e3)

    quantized = jnp.transpose(quant_flat.reshape(B, T, C), (0, 2, 1))
    codes = codes_raw.reshape(N_Q, B, T)
    sub_quants = jnp.transpose(subq_raw.reshape(N_Q, B, T, C), (0, 1, 3, 2))
    commit_loss = (COMMIT_W / (NTOK * C)) * closs_raw[0]
    penalty = jnp.sum(commit_loss) / jnp.maximum(jnp.sum(commit_loss != 0), 1)
    bw_per_q = math.log2(BINS) * jnp.asarray(sample_rate, dtype=jnp.float32) / HOP
    bw = (N_Q * bw_per_q).astype(x.dtype)
    return (quantized, codes, bw, penalty, sub_quants, last_dist)


# speedup vs baseline: 3.1238x; 1.1041x over previous
import math

import jax
import jax.numpy as jnp
from jax.experimental import pallas as pl
from jax.experimental.pallas import tpu as pltpu

N_Q = 8
BINS = 1024
DIM = 128
HOP = 320
COMMIT_W = 0.1

TT = 1024


def _rvq_body(x_ref, cb_ref, cbt_ref, e3_ref,
              quant_ref, codes_ref, subq_ref, dist_ref, closs_ref):
    r = x_ref[...]
    qacc = jnp.zeros_like(r)
    cacc = jnp.zeros((1, N_Q), jnp.float32)
    qpos = jax.lax.broadcasted_iota(jnp.int32, (1, N_Q), 1)
    iota_f = jax.lax.broadcasted_iota(jnp.int32, (TT, BINS), 1).astype(jnp.float32)

    @pl.when(pl.program_id(0) == 0)
    def _():
        closs_ref[...] = jnp.zeros_like(closs_ref)

    for q in range(N_Q):
        emb = cb_ref[q]
        e2 = jnp.sum(emb * emb, axis=1)[None, :]
        r2 = jnp.sum(r * r, axis=1, keepdims=True)
        rb2 = (r + r).astype(jnp.bfloat16)
        s2 = jnp.dot(rb2, cbt_ref[q], preferred_element_type=jnp.float32)
        t = r2 - s2 + e2
        m = jnp.min(t, axis=1, keepdims=True)
        idxf = jnp.min(jnp.where(t == m, iota_f, float(BINS)), axis=1)
        oh = (iota_f == idxf[:, None]).astype(jnp.float32)
        idx = idxf.astype(jnp.int32)
        q3 = jnp.dot(oh, e3_ref[q], preferred_element_type=jnp.float32)
        quant = (q3[:, :DIM] + q3[:, DIM:2 * DIM]) + q3[:, 2 * DIM:]
        diff = quant - r
        cacc = cacc + jnp.sum(diff * diff) * (qpos == q).astype(jnp.float32)
        qacc = qacc + quant
        r = r - quant
        codes_ref[q, 0, :, :] = idx[None, :]
        subq_ref[q, 0] = quant
        if q == N_Q - 1:
            dist_ref[...] = -t
    closs_ref[...] += cacc
    quant_ref[...] = qacc


def _trunc_bf16(a):
    return jax.lax.bitcast_convert_type(
        jax.lax.bitcast_convert_type(a, jnp.uint32) & jnp.uint32(0xFFFF0000),
        jnp.float32)


def _split3(a):
    hi = _trunc_bf16(a)
    rem = a - hi
    mid = _trunc_bf16(rem)
    lo = rem - mid
    return hi, mid, lo


def kernel(x, sample_rate, codebooks):
    B, C, T = x.shape
    NTOK = B * T
    NT = NTOK // TT

    flat = jnp.transpose(x, (0, 2, 1)).reshape(NTOK, C)
    cbt = jnp.transpose(codebooks, (0, 2, 1)).astype(jnp.bfloat16)
    e3 = jnp.concatenate(_split3(codebooks), axis=-1)

    grid = (NT,)
    out_shapes = (
        jax.ShapeDtypeStruct((NTOK, DIM), jnp.float32),
        jax.ShapeDtypeStruct((N_Q, NT, 1, TT), jnp.int32),
        jax.ShapeDtypeStruct((N_Q, NT, TT, DIM), jnp.float32),
        jax.ShapeDtypeStruct((NTOK, BINS), jnp.float32),
        jax.ShapeDtypeStruct((1, N_Q), jnp.float32),
    )
    in_specs = [
        pl.BlockSpec((TT, DIM), lambda i: (i, 0)),
        pl.BlockSpec((N_Q, BINS, DIM), lambda i: (0, 0, 0)),
        pl.BlockSpec((N_Q, DIM, BINS), lambda i: (0, 0, 0)),
        pl.BlockSpec((N_Q, BINS, 3 * DIM), lambda i: (0, 0, 0)),
    ]
    out_specs = (
        pl.BlockSpec((TT, DIM), lambda i: (i, 0)),
        pl.BlockSpec((N_Q, 1, 1, TT), lambda i: (0, i, 0, 0)),
        pl.BlockSpec((N_Q, 1, TT, DIM), lambda i: (0, i, 0, 0)),
        pl.BlockSpec((TT, BINS), lambda i: (i, 0)),
        pl.BlockSpec((1, N_Q), lambda i: (0, 0)),
    )
    quant_flat, codes_raw, subq_raw, last_dist, closs_raw = pl.pallas_call(
        _rvq_body,
        grid=grid,
        in_specs=in_specs,
        out_specs=out_specs,
        out_shape=out_shapes,
        compiler_params=pltpu.CompilerParams(
            dimension_semantics=("arbitrary",),
        ),
    )(flat, codebooks, cbt, e3)

    quantized = jnp.transpose(quant_flat.reshape(B, T, C), (0, 2, 1))
    codes = codes_raw.reshape(N_Q, B, T)
    sub_quants = jnp.transpose(subq_raw.reshape(N_Q, B, T, C), (0, 1, 3, 2))
    commit_loss = (COMMIT_W / (NTOK * C)) * closs_raw[0]
    penalty = jnp.sum(commit_loss) / jnp.maximum(jnp.sum(commit_loss != 0), 1)
    bw_per_q = math.log2(BINS) * jnp.asarray(sample_rate, dtype=jnp.float32) / HOP
    bw = (N_Q * bw_per_q).astype(x.dtype)
    return (quantized, codes, bw, penalty, sub_quants, last_dist)
